# baseline (speedup 1.0000x reference)
import functools

import jax
import jax.numpy as jnp
from jax import lax
from jax.experimental import pallas as pl
from jax.experimental.pallas import tpu as pltpu
from jax.experimental.pallas import tpu_sc as plsc

BATCH = 16384
HIST = 50
EMBED_DIM = 64
VOCAB = 1000

K = 4
CPB = 8


def _build():
    info = plsc.get_sparse_core_info()
    nw = info.num_cores * info.num_subcores
    b_per_w = BATCH // nw
    n_chunks = b_per_w // K
    n_bodies = n_chunks // CPB
    assert b_per_w % K == 0 and n_chunks % CPB == 0

    mesh = plsc.VectorSubcoreMesh(core_axis_name="c", subcore_axis_name="s")

    @functools.partial(
        pl.kernel,
        mesh=mesh,
        out_type=jax.ShapeDtypeStruct((BATCH, HIST, EMBED_DIM), jnp.float32),
        scratch_types=[
            pltpu.VMEM((CPB, K, HIST), jnp.int32),
            pltpu.VMEM((2, K, HIST, EMBED_DIM), jnp.float32),
            [pltpu.SemaphoreType.DMA] * CPB,
            [pltpu.SemaphoreType.DMA] * 2,
            [pltpu.SemaphoreType.DMA] * 2,
        ],
        compiler_params=pltpu.CompilerParams(use_tc_tiling_on_sc=False),
    )
    def kern(idx_hbm, table_hbm, out_hbm, idx_v, rows_v, isems, gsems, osems):
        wid = lax.axis_index("s") * info.num_cores + lax.axis_index("c")
        b0 = wid * b_per_w

        def body(r, carry):
            base = b0 + r * (CPB * K)

            idx_cp = [
                pltpu.async_copy(
                    idx_hbm.at[pl.ds(base + j * K, K)], idx_v.at[j], isems[j]
                )
                for j in range(CPB)
            ]

            def fire_gathers(j, s):
                return [
                    pltpu.async_copy(
                        table_hbm.at[idx_v.at[j].at[q]],
                        rows_v.at[s].at[q],
                        gsems[s],
                    )
                    for q in range(K)
                ]

            def fire_write(j, s):
                return pltpu.async_copy(
                    rows_v.at[s], out_hbm.at[pl.ds(base + j * K, K)], osems[s]
                )

            idx_cp[0].wait()
            g = fire_gathers(0, 0)
            writes = []
            for j in range(CPB):
                s = j % 2
                for c in g:
                    c.wait()
                writes.append(fire_write(j, s))
                if j + 1 < CPB:
                    idx_cp[j + 1].wait()
                    if j >= 1:
                        writes[j - 1].wait()
                    g = fire_gathers(j + 1, 1 - s)
            writes[CPB - 2].wait()
            writes[CPB - 1].wait()
            return carry

        lax.fori_loop(0, n_bodies, body, 0)

    return kern


_kern = _build()


REP = 8


def kernel(indices, table):
    table_rep = jnp.tile(table, (REP, 1))
    off = (jnp.arange(BATCH, dtype=jnp.int32) % REP)[:, None] * VOCAB
    idx_off = indices.astype(jnp.int32) + off
    return _kern(idx_off, table_rep)
